# baseline (speedup 1.0000x reference)
import functools

import jax
import jax.numpy as jnp
from jax import lax
from jax.experimental import pallas as pl
from jax.experimental.pallas import tpu as pltpu
from jax.experimental.pallas import tpu_sc as plsc

BATCH = 16384
EMBED_DIM = 32
NUM_WORKERS = 32
B_PER_W = BATCH // NUM_WORKERS
UNROLL = 16

MLP_BLK = 2048


CHUNK = 128
NCHUNK = B_PER_W // CHUNK

DT_COLS = 40960


DT_Q = DT_COLS // 4


def _detile_body(xT_ref, out_ref):
    x = xT_ref[...]
    xbig = jnp.concatenate(
        [x[:, j * DT_Q:(j + 1) * DT_Q] for j in range(4)], axis=0)
    eye = (lax.broadcasted_iota(jnp.int32, (128, 128), 0)
           == lax.broadcasted_iota(jnp.int32, (128, 128), 1)).astype(jnp.float32)
    out_ref[...] = lax.dot_general(xbig, eye, (((0,), (0,)), ((), ())),
                                   preferred_element_type=jnp.float32)


def _detile_call(n_rows):
    grid = pl.cdiv(n_rows, DT_COLS)
    return pl.pallas_call(
        _detile_body,
        grid=(grid,),
        in_specs=[pl.BlockSpec((EMBED_DIM, DT_COLS), lambda i: (0, i))],
        out_specs=pl.BlockSpec((DT_Q, 4 * EMBED_DIM), lambda i: (i, 0)),
        out_shape=jax.ShapeDtypeStruct((grid * DT_Q, 4 * EMBED_DIM), jnp.float32),
    )


def _slot_transform(q):
    i = lax.div(q, DT_COLS)
    t = q - i * DT_COLS
    j = lax.div(t, DT_Q)
    r = t - j * DT_Q
    return i * DT_COLS + r * 4 + j


def _gather_body(ids_hbm, emb_hbm, out_hbm, idx_v, rows_v, sem):
    wid = lax.axis_index("s") * 2 + lax.axis_index("c")
    base = wid * B_PER_W
    for j in range(NCHUNK):
        pltpu.sync_copy(ids_hbm.at[pl.ds(base + j * CHUNK, CHUNK)], idx_v.at[j])
    for j in range(NCHUNK):
        row = idx_v.at[j]
        for k in range(CHUNK // 16):
            sl = pl.ds(k * 16, 16)
            row[sl] = _slot_transform(row[sl])
    copies = []
    for j in range(NCHUNK):
        copies.append(pltpu.async_copy(emb_hbm.at[idx_v.at[j]], rows_v.at[j], sem))
    for c in copies:
        c.wait()
    for j in range(NCHUNK):
        pltpu.sync_copy(rows_v.at[j], out_hbm.at[pl.ds(base + j * CHUNK, CHUNK)])


@functools.cache
def _gather_call():
    return functools.partial(
        pl.kernel,
        out_type=jax.ShapeDtypeStruct((BATCH, EMBED_DIM), jnp.float32),
        mesh=plsc.VectorSubcoreMesh(core_axis_name="c", subcore_axis_name="s"),
        scratch_types=[
            pltpu.VMEM((NCHUNK, CHUNK), jnp.int32),
            pltpu.VMEM((NCHUNK, CHUNK, EMBED_DIM), jnp.float32),
            pltpu.SemaphoreType.DMA,
        ],
        compiler_params=pltpu.CompilerParams(use_tc_tiling_on_sc=False),
    )(_gather_body)


def _mlp_body(u_ref, v_ref, w1_ref, b1_ref, w2_ref, b2_ref,
              w3_ref, b3_ref, w4_ref, b4_ref, out_ref):
    w1 = w1_ref[...]
    x = jnp.dot(u_ref[...], w1[:EMBED_DIM], preferred_element_type=jnp.float32)
    x = x + jnp.dot(v_ref[...], w1[EMBED_DIM:], preferred_element_type=jnp.float32)
    h = jnp.maximum(x + b1_ref[...].reshape(1, 128), 0.0)
    h = jnp.maximum(
        jnp.dot(h, w2_ref[...], preferred_element_type=jnp.float32)
        + b2_ref[...].reshape(1, 64), 0.0)
    h = jnp.maximum(
        jnp.dot(h, w3_ref[...], preferred_element_type=jnp.float32)
        + b3_ref[...].reshape(1, 32), 0.0)
    w4t = w4_ref[...].reshape(1, 32)
    out_ref[...] = jnp.sum(h * w4t, axis=1) + b4_ref[...]


def _full(shape):
    return pl.BlockSpec(shape, lambda i: tuple(0 for _ in shape))


_mlp_call = pl.pallas_call(
    _mlp_body,
    grid=(BATCH // MLP_BLK,),
    in_specs=[
        pl.BlockSpec((MLP_BLK, EMBED_DIM), lambda i: (i, 0)),
        pl.BlockSpec((MLP_BLK, EMBED_DIM), lambda i: (i, 0)),
        _full((2 * EMBED_DIM, 128)),
        _full((128,)),
        _full((128, 64)),
        _full((64,)),
        _full((64, 32)),
        _full((32,)),
        _full((32, 1)),
        _full((1,)),
    ],
    out_specs=pl.BlockSpec((MLP_BLK,), lambda i: (i,)),
    out_shape=jax.ShapeDtypeStruct((BATCH,), jnp.float32),
)


def kernel(user_ids, item_ids, user_emb, item_emb, W1, b1, W2, b2, W3, b3, W4, b4):
    n_users, n_items = user_emb.shape[0], item_emb.shape[0]
    i_tab = _detile_call(n_items)(item_emb.T).reshape(-1, EMBED_DIM)
    v = _gather_call()(item_ids.astype(jnp.int32), i_tab)
    u_tab = _detile_call(n_users)(user_emb.T).reshape(-1, EMBED_DIM)
    u = _gather_call()(user_ids.astype(jnp.int32), u_tab)
    return _mlp_call(u, v, W1, b1, W2, b2, W3, b3, W4, b4)


# speedup vs baseline: 1.0034x; 1.0034x over previous
import functools

import jax
import jax.numpy as jnp
from jax import lax
from jax.experimental import pallas as pl
from jax.experimental.pallas import tpu as pltpu
from jax.experimental.pallas import tpu_sc as plsc

BATCH = 16384
EMBED_DIM = 32
NUM_WORKERS = 32
B_PER_W = BATCH // NUM_WORKERS
UNROLL = 16

MLP_BLK = 2048


CHUNK = 128
NCHUNK = B_PER_W // CHUNK

DT_COLS = 20480


DT_Q = DT_COLS // 4


def _detile_body(xT_ref, out_ref):
    x = xT_ref[...]
    xbig = jnp.concatenate(
        [x[:, j * DT_Q:(j + 1) * DT_Q] for j in range(4)], axis=0)
    eye = (lax.broadcasted_iota(jnp.int32, (128, 128), 0)
           == lax.broadcasted_iota(jnp.int32, (128, 128), 1)).astype(jnp.float32)
    out_ref[...] = lax.dot_general(xbig, eye, (((0,), (0,)), ((), ())),
                                   preferred_element_type=jnp.float32)


def _detile_call(n_rows):
    grid = pl.cdiv(n_rows, DT_COLS)
    return pl.pallas_call(
        _detile_body,
        grid=(grid,),
        in_specs=[pl.BlockSpec((EMBED_DIM, DT_COLS), lambda i: (0, i))],
        out_specs=pl.BlockSpec((DT_Q, 4 * EMBED_DIM), lambda i: (i, 0)),
        out_shape=jax.ShapeDtypeStruct((grid * DT_Q, 4 * EMBED_DIM), jnp.float32),
    )


def _slot_transform(q):
    i = lax.div(q, DT_COLS)
    t = q - i * DT_COLS
    j = lax.div(t, DT_Q)
    r = t - j * DT_Q
    return i * DT_COLS + r * 4 + j


def _gather_body(ids_hbm, emb_hbm, out_hbm, idx_v, rows_v, sem):
    wid = lax.axis_index("s") * 2 + lax.axis_index("c")
    base = wid * B_PER_W
    for j in range(NCHUNK):
        pltpu.sync_copy(ids_hbm.at[pl.ds(base + j * CHUNK, CHUNK)], idx_v.at[j])
    for j in range(NCHUNK):
        row = idx_v.at[j]
        for k in range(CHUNK // 16):
            sl = pl.ds(k * 16, 16)
            row[sl] = _slot_transform(row[sl])
    copies = []
    for j in range(NCHUNK):
        copies.append(pltpu.async_copy(emb_hbm.at[idx_v.at[j]], rows_v.at[j], sem))
    for c in copies:
        c.wait()
    for j in range(NCHUNK):
        pltpu.sync_copy(rows_v.at[j], out_hbm.at[pl.ds(base + j * CHUNK, CHUNK)])


@functools.cache
def _gather_call():
    return functools.partial(
        pl.kernel,
        out_type=jax.ShapeDtypeStruct((BATCH, EMBED_DIM), jnp.float32),
        mesh=plsc.VectorSubcoreMesh(core_axis_name="c", subcore_axis_name="s"),
        scratch_types=[
            pltpu.VMEM((NCHUNK, CHUNK), jnp.int32),
            pltpu.VMEM((NCHUNK, CHUNK, EMBED_DIM), jnp.float32),
            pltpu.SemaphoreType.DMA,
        ],
        compiler_params=pltpu.CompilerParams(use_tc_tiling_on_sc=False),
    )(_gather_body)


def _mlp_body(u_ref, v_ref, w1_ref, b1_ref, w2_ref, b2_ref,
              w3_ref, b3_ref, w4_ref, b4_ref, out_ref):
    w1 = w1_ref[...]
    x = jnp.dot(u_ref[...], w1[:EMBED_DIM], preferred_element_type=jnp.float32)
    x = x + jnp.dot(v_ref[...], w1[EMBED_DIM:], preferred_element_type=jnp.float32)
    h = jnp.maximum(x + b1_ref[...].reshape(1, 128), 0.0)
    h = jnp.maximum(
        jnp.dot(h, w2_ref[...], preferred_element_type=jnp.float32)
        + b2_ref[...].reshape(1, 64), 0.0)
    h = jnp.maximum(
        jnp.dot(h, w3_ref[...], preferred_element_type=jnp.float32)
        + b3_ref[...].reshape(1, 32), 0.0)
    w4t = w4_ref[...].reshape(1, 32)
    out_ref[...] = jnp.sum(h * w4t, axis=1) + b4_ref[...]


def _full(shape):
    return pl.BlockSpec(shape, lambda i: tuple(0 for _ in shape))


_mlp_call = pl.pallas_call(
    _mlp_body,
    grid=(BATCH // MLP_BLK,),
    in_specs=[
        pl.BlockSpec((MLP_BLK, EMBED_DIM), lambda i: (i, 0)),
        pl.BlockSpec((MLP_BLK, EMBED_DIM), lambda i: (i, 0)),
        _full((2 * EMBED_DIM, 128)),
        _full((128,)),
        _full((128, 64)),
        _full((64,)),
        _full((64, 32)),
        _full((32,)),
        _full((32, 1)),
        _full((1,)),
    ],
    out_specs=pl.BlockSpec((MLP_BLK,), lambda i: (i,)),
    out_shape=jax.ShapeDtypeStruct((BATCH,), jnp.float32),
)


def kernel(user_ids, item_ids, user_emb, item_emb, W1, b1, W2, b2, W3, b3, W4, b4):
    n_users, n_items = user_emb.shape[0], item_emb.shape[0]
    i_tab = _detile_call(n_items)(item_emb.T).reshape(-1, EMBED_DIM)
    v = _gather_call()(item_ids.astype(jnp.int32), i_tab)
    u_tab = _detile_call(n_users)(user_emb.T).reshape(-1, EMBED_DIM)
    u = _gather_call()(user_ids.astype(jnp.int32), u_tab)
    return _mlp_call(u, v, W1, b1, W2, b2, W3, b3, W4, b4)
